# baseline (device time: 67222 ns/iter reference)
import jax
import jax.numpy as jnp
from jax import lax
from jax.experimental import pallas as pl
from jax.experimental.pallas import tpu as pltpu

M, N = 2048, 1024
MESH = pl.DeviceIdType.MESH

SCHEDULES = [
    (0, 704, ("x", "y", "z")),
    (704, 704, ("y", "z", "x")),
    (1408, 640, ("z", "x", "y")),
]

SEMS_PER_PART = 14


def kernel(x):
    def body(x_ref, out_ref, scratch, send_sems, recv_sems):
        mx = lax.axis_index("x")
        my = lax.axis_index("y")
        mz = lax.axis_index("z")
        bits = {"x": mx, "y": my, "z": mz}
        partner = {
            "x": (1 - mx, my, mz),
            "y": (mx, 1 - my, mz),
            "z": (mx, my, 1 - mz),
        }

        barrier = pltpu.get_barrier_semaphore()
        for ax in ("x", "y", "z"):
            pl.semaphore_signal(
                barrier, inc=1, device_id=partner[ax], device_id_type=MESH
            )
        pl.semaphore_wait(barrier, 3)

        def blk_off(p_idx, flips=()):
            off, rows, order = SCHEDULES[p_idx]
            k = off
            for j, ax in enumerate(order):
                b = (1 - bits[ax]) if ax in flips else bits[ax]
                k = k + b * (rows >> (j + 1))
            return k

        geoms = []
        scr_base = 0
        for off, rows, order in SCHEDULES:
            b0, b1, b2 = (bits[a] for a in order)
            h, q, e = rows >> 1, rows >> 2, rows >> 3
            k1 = off + b0 * h
            k2 = k1 + b1 * q
            k3 = k2 + b2 * e
            send0 = off + (1 - b0) * h
            s1 = k1 + (1 - b1) * q
            s2 = k2 + (1 - b2) * e
            r_s1 = (1 - b1) * q
            r_s2 = b1 * q + (1 - b2) * e
            r_k = b1 * q + b2 * e
            r1_s2 = (1 - b2) * e
            r1_k = b2 * e
            geoms.append(
                dict(
                    order=order, h=h, q=q, e=e,
                    k1=k1, k2=k2, k3=k3, send0=send0, s1=s1, s2=s2,
                    r_s1=r_s1, r_s2=r_s2, r_k=r_k, r1_s2=r1_s2, r1_k=r1_k,
                    scr0=scr_base, scr1=scr_base + h, scr2=scr_base + h + q,
                )
            )
            scr_base += h + q + e

        inflight = []

        def copy(p_idx, slot, ax, src_ref, s0, dst_ref, d0, sz):
            rdma = pltpu.make_async_remote_copy(
                src_ref=src_ref.at[pl.ds(s0, sz), :],
                dst_ref=dst_ref.at[pl.ds(d0, sz), :],
                send_sem=send_sems.at[p_idx * SEMS_PER_PART + slot],
                recv_sem=recv_sems.at[p_idx * SEMS_PER_PART + slot],
                device_id=partner[ax],
                device_id_type=MESH,
            )
            rdma.start()
            inflight.append(rdma)
            return rdma

        def send_blk(p_idx, flips, ax, slot):
            e = geoms[p_idx]["e"]
            k0 = blk_off(p_idx, flips)
            return copy(p_idx, slot, ax, out_ref, k0, out_ref, k0, e)

        def bcast_axes(p_idx):
            order = SCHEDULES[p_idx][2]
            return order[2], order[1], order[0]

        r0 = []
        for p_idx, g in enumerate(geoms):
            a0 = g["order"][0]
            chunks = []
            for slot, rel in (
                (0, g["r_s1"] + g["r1_s2"]),
                (1, g["r_s1"] + g["r1_k"]),
                (2, g["r_s2"]),
                (3, g["r_k"]),
            ):
                chunks.append(
                    copy(p_idx, slot, a0, x_ref, g["send0"] + rel,
                         scratch, g["scr0"] + rel, g["e"])
                )
            r0.append(chunks)

        r1 = []
        for p_idx, g in enumerate(geoms):
            a1 = g["order"][1]
            r0[p_idx][0].wait_recv()
            rel = g["r_s1"] + g["r1_s2"]
            out_ref[pl.ds(g["s1"] + g["r1_s2"], g["e"]), :] = (
                x_ref[pl.ds(g["s1"] + g["r1_s2"], g["e"]), :]
                + scratch[pl.ds(g["scr0"] + rel, g["e"]), :]
            )
            r1.append([
                copy(p_idx, 4, a1, out_ref, g["s1"] + g["r1_s2"],
                     scratch, g["scr1"] + g["r1_s2"], g["e"])
            ])
        for p_idx, g in enumerate(geoms):
            a1 = g["order"][1]
            r0[p_idx][1].wait_recv()
            rel = g["r_s1"] + g["r1_k"]
            out_ref[pl.ds(g["s1"] + g["r1_k"], g["e"]), :] = (
                x_ref[pl.ds(g["s1"] + g["r1_k"], g["e"]), :]
                + scratch[pl.ds(g["scr0"] + rel, g["e"]), :]
            )
            r1[p_idx].append(
                copy(p_idx, 5, a1, out_ref, g["s1"] + g["r1_k"],
                     scratch, g["scr1"] + g["r1_k"], g["e"])
            )

        r2 = []
        for p_idx, g in enumerate(geoms):
            a2 = g["order"][2]
            r1[p_idx][0].wait_recv()
            r0[p_idx][2].wait_recv()
            out_ref[pl.ds(g["s2"], g["e"]), :] = (
                x_ref[pl.ds(g["s2"], g["e"]), :]
                + scratch[pl.ds(g["scr0"] + g["r_s2"], g["e"]), :]
                + scratch[pl.ds(g["scr1"] + g["r1_s2"], g["e"]), :]
            )
            r2.append(
                copy(p_idx, 6, a2, out_ref, g["s2"], scratch, g["scr2"],
                     g["e"])
            )

        ag0 = []
        for p_idx, g in enumerate(geoms):
            r2[p_idx].wait_recv()
            r0[p_idx][3].wait_recv()
            r1[p_idx][1].wait_recv()
            out_ref[pl.ds(g["k3"], g["e"]), :] = (
                x_ref[pl.ds(g["k3"], g["e"]), :]
                + scratch[pl.ds(g["scr0"] + g["r_k"], g["e"]), :]
                + scratch[pl.ds(g["scr1"] + g["r1_k"], g["e"]), :]
                + scratch[pl.ds(g["scr2"], g["e"]), :]
            )
            s0, s1, s2 = bcast_axes(p_idx)
            ag0.append(send_blk(p_idx, (), s0, 7))
            send_blk(p_idx, (), s1, 8)
            send_blk(p_idx, (), s2, 10)

        for p_idx in range(3):
            s0, s1, s2 = bcast_axes(p_idx)
            ag0[p_idx].wait_recv()
            send_blk(p_idx, (s0,), s1, 9)
            send_blk(p_idx, (s0,), s2, 11)
        for p_idx, g in enumerate(geoms):
            s0, s1, s2 = bcast_axes(p_idx)
            for slot, flips, fwd_slot in ((8, (s1,), 12), (9, (s1, s0), 13)):
                k0 = blk_off(p_idx, flips)
                r = pltpu.make_async_remote_copy(
                    src_ref=out_ref.at[pl.ds(k0, g["e"]), :],
                    dst_ref=out_ref.at[pl.ds(k0, g["e"]), :],
                    send_sem=send_sems.at[p_idx * SEMS_PER_PART + slot],
                    recv_sem=recv_sems.at[p_idx * SEMS_PER_PART + slot],
                    device_id=partner[s1],
                    device_id_type=MESH,
                )
                r.wait_recv()
                send_blk(p_idx, flips, s2, fwd_slot)
        for p_idx, g in enumerate(geoms):
            s0, s1, s2 = bcast_axes(p_idx)
            for slot, flips in (
                (10, (s2,)),
                (11, (s2, s0)),
                (12, (s2, s1)),
                (13, (s2, s1, s0)),
            ):
                k0 = blk_off(p_idx, flips)
                r = pltpu.make_async_remote_copy(
                    src_ref=out_ref.at[pl.ds(k0, g["e"]), :],
                    dst_ref=out_ref.at[pl.ds(k0, g["e"]), :],
                    send_sem=send_sems.at[p_idx * SEMS_PER_PART + slot],
                    recv_sem=recv_sems.at[p_idx * SEMS_PER_PART + slot],
                    device_id=partner[s2],
                    device_id_type=MESH,
                )
                r.wait_recv()

        for rdma in inflight:
            rdma.wait_send()

    out_shape = jax.ShapeDtypeStruct((M, N), jnp.float32)
    return pl.pallas_call(
        body,
        out_shape=out_shape,
        in_specs=[pl.BlockSpec(memory_space=pltpu.VMEM)],
        out_specs=pl.BlockSpec(memory_space=pltpu.VMEM),
        scratch_shapes=[
            pltpu.VMEM((1792, N), jnp.float32),
            pltpu.SemaphoreType.DMA((3 * SEMS_PER_PART,)),
            pltpu.SemaphoreType.DMA((3 * SEMS_PER_PART,)),
        ],
        compiler_params=pltpu.CompilerParams(collective_id=0),
    )(x.reshape(M, N))


# device time: 67220 ns/iter; 1.0000x vs baseline; 1.0000x over previous
import jax
import jax.numpy as jnp
from jax import lax
from jax.experimental import pallas as pl
from jax.experimental.pallas import tpu as pltpu

M, N = 2048, 1024
MESH = pl.DeviceIdType.MESH

SCHEDULES = [
    (0, 704, ("x", "y", "z")),
    (704, 704, ("y", "z", "x")),
    (1408, 640, ("z", "x", "y")),
]

SEMS_PER_PART = 14


def kernel(x):
    def body(x_ref, out_ref, scratch, send_sems, recv_sems):
        mx = lax.axis_index("x")
        my = lax.axis_index("y")
        mz = lax.axis_index("z")
        bits = {"x": mx, "y": my, "z": mz}
        partner = {
            "x": (1 - mx, my, mz),
            "y": (mx, 1 - my, mz),
            "z": (mx, my, 1 - mz),
        }

        barrier = pltpu.get_barrier_semaphore()
        for ax in ("x", "y", "z"):
            pl.semaphore_signal(
                barrier, inc=1, device_id=partner[ax], device_id_type=MESH
            )
        pl.semaphore_wait(barrier, 3)

        def blk_off(p_idx, flips=()):
            off, rows, order = SCHEDULES[p_idx]
            k = off
            for j, ax in enumerate(order):
                b = (1 - bits[ax]) if ax in flips else bits[ax]
                k = k + b * (rows >> (j + 1))
            return k

        geoms = []
        scr_base = 0
        for off, rows, order in SCHEDULES:
            b0, b1, b2 = (bits[a] for a in order)
            h, q, e = rows >> 1, rows >> 2, rows >> 3
            k1 = off + b0 * h
            k2 = k1 + b1 * q
            k3 = k2 + b2 * e
            send0 = off + (1 - b0) * h
            s1 = k1 + (1 - b1) * q
            s2 = k2 + (1 - b2) * e
            r_s1 = (1 - b1) * q
            r_s2 = b1 * q + (1 - b2) * e
            r_k = b1 * q + b2 * e
            r1_s2 = (1 - b2) * e
            r1_k = b2 * e
            geoms.append(
                dict(
                    order=order, h=h, q=q, e=e,
                    k1=k1, k2=k2, k3=k3, send0=send0, s1=s1, s2=s2,
                    r_s1=r_s1, r_s2=r_s2, r_k=r_k, r1_s2=r1_s2, r1_k=r1_k,
                    scr0=scr_base, scr1=scr_base + h, scr2=scr_base + h + q,
                )
            )
            scr_base += h + q + e

        inflight = []

        def copy(p_idx, slot, ax, src_ref, s0, dst_ref, d0, sz):
            rdma = pltpu.make_async_remote_copy(
                src_ref=src_ref.at[pl.ds(s0, sz), :],
                dst_ref=dst_ref.at[pl.ds(d0, sz), :],
                send_sem=send_sems.at[p_idx * SEMS_PER_PART + slot],
                recv_sem=recv_sems.at[p_idx * SEMS_PER_PART + slot],
                device_id=partner[ax],
                device_id_type=MESH,
            )
            rdma.start()
            inflight.append(rdma)
            return rdma

        def send_blk(p_idx, flips, ax, slot):
            e = geoms[p_idx]["e"]
            k0 = blk_off(p_idx, flips)
            return copy(p_idx, slot, ax, out_ref, k0, out_ref, k0, e)

        def bcast_axes(p_idx):
            order = SCHEDULES[p_idx][2]
            return order[2], order[1], order[0]

        def start_r0(p_idx, c):
            g = geoms[p_idx]
            rel = (
                g["r_s1"] + g["r1_s2"],
                g["r_s1"] + g["r1_k"],
                g["r_s2"],
                g["r_k"],
            )[c]
            return copy(p_idx, c, g["order"][0], x_ref, g["send0"] + rel,
                        scratch, g["scr0"] + rel, g["e"])

        r0 = [[start_r0(p, 0), start_r0(p, 1)] for p in range(3)]

        r1 = []
        for p_idx, g in enumerate(geoms):
            a1 = g["order"][1]
            r0[p_idx][0].wait_recv()
            rel = g["r_s1"] + g["r1_s2"]
            out_ref[pl.ds(g["s1"] + g["r1_s2"], g["e"]), :] = (
                x_ref[pl.ds(g["s1"] + g["r1_s2"], g["e"]), :]
                + scratch[pl.ds(g["scr0"] + rel, g["e"]), :]
            )
            r1.append([
                copy(p_idx, 4, a1, out_ref, g["s1"] + g["r1_s2"],
                     scratch, g["scr1"] + g["r1_s2"], g["e"])
            ])
            r0[p_idx].append(start_r0(p_idx, 2))
        for p_idx, g in enumerate(geoms):
            a1 = g["order"][1]
            r0[p_idx][1].wait_recv()
            rel = g["r_s1"] + g["r1_k"]
            out_ref[pl.ds(g["s1"] + g["r1_k"], g["e"]), :] = (
                x_ref[pl.ds(g["s1"] + g["r1_k"], g["e"]), :]
                + scratch[pl.ds(g["scr0"] + rel, g["e"]), :]
            )
            r1[p_idx].append(
                copy(p_idx, 5, a1, out_ref, g["s1"] + g["r1_k"],
                     scratch, g["scr1"] + g["r1_k"], g["e"])
            )
            r0[p_idx].append(start_r0(p_idx, 3))

        r2 = []
        for p_idx, g in enumerate(geoms):
            a2 = g["order"][2]
            r1[p_idx][0].wait_recv()
            r0[p_idx][2].wait_recv()
            out_ref[pl.ds(g["s2"], g["e"]), :] = (
                x_ref[pl.ds(g["s2"], g["e"]), :]
                + scratch[pl.ds(g["scr0"] + g["r_s2"], g["e"]), :]
                + scratch[pl.ds(g["scr1"] + g["r1_s2"], g["e"]), :]
            )
            r2.append(
                copy(p_idx, 6, a2, out_ref, g["s2"], scratch, g["scr2"],
                     g["e"])
            )

        ag0 = []
        for p_idx, g in enumerate(geoms):
            r2[p_idx].wait_recv()
            r0[p_idx][3].wait_recv()
            r1[p_idx][1].wait_recv()
            out_ref[pl.ds(g["k3"], g["e"]), :] = (
                x_ref[pl.ds(g["k3"], g["e"]), :]
                + scratch[pl.ds(g["scr0"] + g["r_k"], g["e"]), :]
                + scratch[pl.ds(g["scr1"] + g["r1_k"], g["e"]), :]
                + scratch[pl.ds(g["scr2"], g["e"]), :]
            )
            s0, s1, s2 = bcast_axes(p_idx)
            ag0.append(send_blk(p_idx, (), s0, 7))
            send_blk(p_idx, (), s1, 8)
            send_blk(p_idx, (), s2, 10)

        for p_idx in range(3):
            s0, s1, s2 = bcast_axes(p_idx)
            ag0[p_idx].wait_recv()
            send_blk(p_idx, (s0,), s1, 9)
            send_blk(p_idx, (s0,), s2, 11)
        for p_idx, g in enumerate(geoms):
            s0, s1, s2 = bcast_axes(p_idx)
            for slot, flips, fwd_slot in ((8, (s1,), 12), (9, (s1, s0), 13)):
                k0 = blk_off(p_idx, flips)
                r = pltpu.make_async_remote_copy(
                    src_ref=out_ref.at[pl.ds(k0, g["e"]), :],
                    dst_ref=out_ref.at[pl.ds(k0, g["e"]), :],
                    send_sem=send_sems.at[p_idx * SEMS_PER_PART + slot],
                    recv_sem=recv_sems.at[p_idx * SEMS_PER_PART + slot],
                    device_id=partner[s1],
                    device_id_type=MESH,
                )
                r.wait_recv()
                send_blk(p_idx, flips, s2, fwd_slot)
        for p_idx, g in enumerate(geoms):
            s0, s1, s2 = bcast_axes(p_idx)
            for slot, flips in (
                (10, (s2,)),
                (11, (s2, s0)),
                (12, (s2, s1)),
                (13, (s2, s1, s0)),
            ):
                k0 = blk_off(p_idx, flips)
                r = pltpu.make_async_remote_copy(
                    src_ref=out_ref.at[pl.ds(k0, g["e"]), :],
                    dst_ref=out_ref.at[pl.ds(k0, g["e"]), :],
                    send_sem=send_sems.at[p_idx * SEMS_PER_PART + slot],
                    recv_sem=recv_sems.at[p_idx * SEMS_PER_PART + slot],
                    device_id=partner[s2],
                    device_id_type=MESH,
                )
                r.wait_recv()

        for rdma in inflight:
            rdma.wait_send()

    out_shape = jax.ShapeDtypeStruct((M, N), jnp.float32)
    return pl.pallas_call(
        body,
        out_shape=out_shape,
        in_specs=[pl.BlockSpec(memory_space=pltpu.VMEM)],
        out_specs=pl.BlockSpec(memory_space=pltpu.VMEM),
        scratch_shapes=[
            pltpu.VMEM((1792, N), jnp.float32),
            pltpu.SemaphoreType.DMA((3 * SEMS_PER_PART,)),
            pltpu.SemaphoreType.DMA((3 * SEMS_PER_PART,)),
        ],
        compiler_params=pltpu.CompilerParams(collective_id=0),
    )(x.reshape(M, N))
